# speedup vs baseline: 25.2851x; 25.2851x over previous
import functools
import jax
import jax.numpy as jnp
from jax import lax
from jax.experimental import pallas as pl
from jax.experimental.pallas import tpu as pltpu
from jax.experimental.pallas import tpu_sc as plsc

H = 256
NHI = 81
BAND = 256
NW = 32
CHUNK = 128


def _tbl_body(w0, w1, w2, w3, w4, w5, w6, w7, w8, out):
    cb = pl.program_id(0)
    acc = w0[...]
    for wref in (w1, w2, w3, w4):
        w = wref[...]
        acc = jnp.concatenate(
            [acc + w[0:1, :], acc + w[1:2, :], acc + w[2:3, :]], axis=0)
    acc = jnp.concatenate([acc, jnp.zeros((BAND - 243, H), jnp.float32)],
                          axis=0)
    hi = jnp.zeros((1, H), jnp.float32)
    r = cb
    for wref in (w5, w6, w7, w8):
        w = wref[...]
        d = r % 3
        r = r // 3
        hi = hi + jnp.where(d == 0, w[0:1, :],
                            jnp.where(d == 1, w[1:2, :], w[2:3, :]))
    out[...] = (acc + hi) * 16.0


def _build_table(ws, interpret=False):
    return pl.pallas_call(
        _tbl_body,
        grid=(NHI,),
        in_specs=[pl.BlockSpec((3, H), lambda i: (0, 0))] * 9,
        out_specs=pl.BlockSpec((BAND, H), lambda i: (i, 0)),
        out_shape=jax.ShapeDtypeStruct((NHI * BAND, H), jnp.float32),
        interpret=interpret,
    )(*ws)


def _sc_body(n_tok, i0, i1, i2, i3, i4, i5, i6, i7, i8, table, out,
             b0, b1, b2, b3, b4, b5, b6, b7, b8, codes, rows, sem):
    per_w = n_tok // NW
    nchunk = per_w // CHUNK
    wid = lax.axis_index("s") * 2 + lax.axis_index("c")
    base = wid * per_w
    bufs = (b0, b1, b2, b3, b4, b5, b6, b7, b8)
    for src, dst in zip((i0, i1, i2, i3, i4, i5, i6, i7, i8), bufs):
        pltpu.sync_copy(src.at[pl.ds(base, per_w)], dst)

    def code_body(j, carry):
        o = j * 16
        d = [b[pl.ds(o, 16)] for b in bufs]
        hi = ((d[8] * 3 + d[7]) * 3 + d[6]) * 3 + d[5]
        lo = (((d[4] * 3 + d[3]) * 3 + d[2]) * 3 + d[1]) * 3 + d[0]
        codes[pl.ds(o, 16)] = hi * BAND + lo
        return carry

    lax.fori_loop(0, per_w // 16, code_body, 0)

    def chunk_body(g, carry):
        cb = g * CHUNK
        pltpu.async_copy(table.at[codes.at[pl.ds(cb, CHUNK)]], rows,
                         sem).wait()
        pltpu.sync_copy(rows, out.at[pl.ds(base + cb, CHUNK)])
        return carry

    lax.fori_loop(0, nchunk, chunk_body, 0)


def _sc_lookup(idxs, table, interpret=False):
    n_tok = idxs[0].shape[0]
    per_w = n_tok // NW
    mesh = plsc.VectorSubcoreMesh(core_axis_name="c", subcore_axis_name="s")
    scratch = [pltpu.VMEM((per_w,), jnp.int32) for _ in range(9)]
    scratch += [
        pltpu.VMEM((per_w,), jnp.int32),
        pltpu.VMEM((CHUNK, H), jnp.float32),
        pltpu.SemaphoreType.DMA,
    ]
    fn = pl.kernel(
        functools.partial(_sc_body, n_tok),
        out_type=jax.ShapeDtypeStruct((n_tok, H), jnp.float32),
        mesh=mesh,
        scratch_types=scratch,
        interpret=interpret,
    )
    return fn(*idxs, table)


def kernel(mix, falsetto, breathy, bubble, strong, weak, pharyngeal,
           vibrato, glissando,
           W_mix, W_falsetto, W_breathy, W_bubble, W_strong, W_weak,
           W_pharyngeal, W_vibrato, W_glissando):
    b, t = mix.shape
    idxs = [a.reshape(-1) for a in
            (mix, falsetto, breathy, bubble, strong, weak, pharyngeal,
             vibrato, glissando)]
    ws = (W_mix, W_falsetto, W_breathy, W_bubble, W_strong, W_weak,
          W_pharyngeal, W_vibrato, W_glissando)
    table = _build_table(ws)
    out = _sc_lookup(idxs, table)
    return out.reshape(b, t, H)


# speedup vs baseline: 29.4884x; 1.1662x over previous
import functools
import jax
import jax.numpy as jnp
from jax import lax
from jax.experimental import pallas as pl
from jax.experimental.pallas import tpu as pltpu
from jax.experimental.pallas import tpu_sc as plsc

H = 256
NHI = 81
BAND = 256
NW = 32
CHUNK = 128


def _tbl_body(w0, w1, w2, w3, w4, w5, w6, w7, w8, out):
    cb = pl.program_id(0)
    acc = w0[...]
    for wref in (w1, w2, w3, w4):
        w = wref[...]
        acc = jnp.concatenate(
            [acc + w[0:1, :], acc + w[1:2, :], acc + w[2:3, :]], axis=0)
    acc = jnp.concatenate([acc, jnp.zeros((BAND - 243, H), jnp.float32)],
                          axis=0)
    hi = jnp.zeros((1, H), jnp.float32)
    r = cb
    for wref in (w5, w6, w7, w8):
        w = wref[...]
        d = r % 3
        r = r // 3
        hi = hi + jnp.where(d == 0, w[0:1, :],
                            jnp.where(d == 1, w[1:2, :], w[2:3, :]))
    out[...] = (acc + hi) * 16.0


def _build_table(ws, interpret=False):
    return pl.pallas_call(
        _tbl_body,
        grid=(NHI,),
        in_specs=[pl.BlockSpec((3, H), lambda i: (0, 0))] * 9,
        out_specs=pl.BlockSpec((BAND, H), lambda i: (i, 0)),
        out_shape=jax.ShapeDtypeStruct((NHI * BAND, H), jnp.float32),
        interpret=interpret,
    )(*ws)


def _sc_body(n_tok, i0, i1, i2, i3, i4, i5, i6, i7, i8, table, out,
             b0, b1, b2, b3, b4, b5, b6, b7, b8, codes, rows0, rows1,
             isem, gsem0, gsem1, ssem0, ssem1):
    per_w = n_tok // NW
    nchunk = per_w // CHUNK
    wid = lax.axis_index("s") * 2 + lax.axis_index("c")
    base = wid * per_w
    bufs = (b0, b1, b2, b3, b4, b5, b6, b7, b8)
    stages = [pltpu.async_copy(src.at[pl.ds(base, per_w)], dst, isem)
              for src, dst in zip((i0, i1, i2, i3, i4, i5, i6, i7, i8), bufs)]
    for h in stages:
        h.wait()

    def code_body(j, carry):
        o = j * 16
        d = [b[pl.ds(o, 16)] for b in bufs]
        hi = ((d[8] * 3 + d[7]) * 3 + d[6]) * 3 + d[5]
        lo = (((d[4] * 3 + d[3]) * 3 + d[2]) * 3 + d[1]) * 3 + d[0]
        codes[pl.ds(o, 16)] = hi * BAND + lo
        return carry

    lax.fori_loop(0, per_w // 16, code_body, 0)

    slots = ((rows0, gsem0, ssem0), (rows1, gsem1, ssem1))

    def gather(g, rows, gsem):
        pltpu.async_copy(table.at[codes.at[pl.ds(g * CHUNK, CHUNK)]],
                         rows, gsem)

    for b, (rows_b, gsem_b, _) in enumerate(slots):
        gather(b, rows_b, gsem_b)

    @pl.loop(0, nchunk, step=2)
    def chunk_loop(k):
        for b, (rows_b, gsem_b, ssem_b) in enumerate(slots):
            g = k + b
            off = g * CHUNK
            pltpu.make_async_copy(
                table.at[codes.at[pl.ds(off, CHUNK)]], rows_b, gsem_b).wait()
            sc = pltpu.async_copy(rows_b, out.at[pl.ds(base + off, CHUNK)],
                                  ssem_b)
            sc.wait()

            @pl.when(g + 2 < nchunk)
            def _():
                gather(g + 2, rows_b, gsem_b)


def _sc_lookup(idxs, table, interpret=False):
    n_tok = idxs[0].shape[0]
    per_w = n_tok // NW
    mesh = plsc.VectorSubcoreMesh(core_axis_name="c", subcore_axis_name="s")
    scratch = [pltpu.VMEM((per_w,), jnp.int32) for _ in range(9)]
    scratch += [
        pltpu.VMEM((per_w,), jnp.int32),
        pltpu.VMEM((CHUNK, H), jnp.float32),
        pltpu.VMEM((CHUNK, H), jnp.float32),
        pltpu.SemaphoreType.DMA,
        pltpu.SemaphoreType.DMA,
        pltpu.SemaphoreType.DMA,
        pltpu.SemaphoreType.DMA,
        pltpu.SemaphoreType.DMA,
    ]
    fn = pl.kernel(
        functools.partial(_sc_body, n_tok),
        out_type=jax.ShapeDtypeStruct((n_tok, H), jnp.float32),
        mesh=mesh,
        scratch_types=scratch,
        interpret=interpret,
    )
    return fn(*idxs, table)


def kernel(mix, falsetto, breathy, bubble, strong, weak, pharyngeal,
           vibrato, glissando,
           W_mix, W_falsetto, W_breathy, W_bubble, W_strong, W_weak,
           W_pharyngeal, W_vibrato, W_glissando):
    b, t = mix.shape
    idxs = [a.reshape(-1) for a in
            (mix, falsetto, breathy, bubble, strong, weak, pharyngeal,
             vibrato, glissando)]
    ws = (W_mix, W_falsetto, W_breathy, W_bubble, W_strong, W_weak,
          W_pharyngeal, W_vibrato, W_glissando)
    table = _build_table(ws)
    out = _sc_lookup(idxs, table)
    return out.reshape(b, t, H)


# speedup vs baseline: 29.6454x; 1.0053x over previous
import functools
import jax
import jax.numpy as jnp
from jax import lax
from jax.experimental import pallas as pl
from jax.experimental.pallas import tpu as pltpu
from jax.experimental.pallas import tpu_sc as plsc

H = 256
NHI = 81
BAND = 256
NW = 32
CHUNK = 64
DEPTH = 4
AHEAD = 2


def _tbl_body(w0, w1, w2, w3, w4, w5, w6, w7, w8, out):
    cb = pl.program_id(0)
    acc = w0[...]
    for wref in (w1, w2, w3, w4):
        w = wref[...]
        acc = jnp.concatenate(
            [acc + w[0:1, :], acc + w[1:2, :], acc + w[2:3, :]], axis=0)
    acc = jnp.concatenate([acc, jnp.zeros((BAND - 243, H), jnp.float32)],
                          axis=0)
    hi = jnp.zeros((1, H), jnp.float32)
    r = cb
    for wref in (w5, w6, w7, w8):
        w = wref[...]
        d = r % 3
        r = r // 3
        hi = hi + jnp.where(d == 0, w[0:1, :],
                            jnp.where(d == 1, w[1:2, :], w[2:3, :]))
    out[...] = (acc + hi) * 16.0


def _build_table(ws, interpret=False):
    return pl.pallas_call(
        _tbl_body,
        grid=(NHI,),
        in_specs=[pl.BlockSpec((3, H), lambda i: (0, 0))] * 9,
        out_specs=pl.BlockSpec((BAND, H), lambda i: (i, 0)),
        out_shape=jax.ShapeDtypeStruct((NHI * BAND, H), jnp.float32),
        interpret=interpret,
    )(*ws)


def _sc_body(n_tok, i0, i1, i2, i3, i4, i5, i6, i7, i8, table, out,
             b0, b1, b2, b3, b4, b5, b6, b7, b8, codes,
             rows0, rows1, rows2, rows3,
             isem, gsem0, gsem1, gsem2, gsem3, ssem0, ssem1, ssem2, ssem3):
    per_w = n_tok // NW
    nchunk = per_w // CHUNK
    wid = lax.axis_index("s") * 2 + lax.axis_index("c")
    base = wid * per_w
    bufs = (b0, b1, b2, b3, b4, b5, b6, b7, b8)
    stages = [pltpu.async_copy(src.at[pl.ds(base, per_w)], dst, isem)
              for src, dst in zip((i0, i1, i2, i3, i4, i5, i6, i7, i8), bufs)]
    for h in stages:
        h.wait()

    rows = (rows0, rows1, rows2, rows3)
    gsem = (gsem0, gsem1, gsem2, gsem3)
    ssem = (ssem0, ssem1, ssem2, ssem3)

    def code_chunk(g):
        for j in range(CHUNK // 16):
            o = g * CHUNK + j * 16
            d = [b[pl.ds(o, 16)] for b in bufs]
            hi = ((d[8] * 3 + d[7]) * 3 + d[6]) * 3 + d[5]
            lo = (((d[4] * 3 + d[3]) * 3 + d[2]) * 3 + d[1]) * 3 + d[0]
            codes[pl.ds(o, 16)] = hi * BAND + lo

    def gather_start(g, s):
        pltpu.async_copy(table.at[codes.at[pl.ds(g * CHUNK, CHUNK)]],
                         rows[s], gsem[s])

    def gather_wait(g, s):
        pltpu.make_async_copy(table.at[codes.at[pl.ds(g * CHUNK, CHUNK)]],
                              rows[s], gsem[s]).wait()

    def scatter_start(g, s):
        pltpu.async_copy(rows[s], out.at[pl.ds(base + g * CHUNK, CHUNK)],
                         ssem[s])

    def scatter_wait(g, s):
        pltpu.make_async_copy(rows[s], out.at[pl.ds(base + g * CHUNK, CHUNK)],
                              ssem[s]).wait()

    for g in range(AHEAD):
        code_chunk(g)
        gather_start(g, g % DEPTH)

    @pl.loop(0, nchunk, step=DEPTH)
    def chunk_loop(k):
        for b in range(DEPTH):
            g = k + b
            s = b
            gather_wait(g, s)
            scatter_start(g, s)
            nxt = g + AHEAD
            s2 = (b + AHEAD) % DEPTH

            @pl.when(nxt < nchunk)
            def _():
                code_chunk(nxt)

                @pl.when(nxt - DEPTH >= 0)
                def _():
                    scatter_wait(nxt - DEPTH, s2)

                gather_start(nxt, s2)

    for s in range(DEPTH):
        scatter_wait(nchunk - DEPTH + s, s)


def _sc_lookup(idxs, table, interpret=False):
    n_tok = idxs[0].shape[0]
    per_w = n_tok // NW
    mesh = plsc.VectorSubcoreMesh(core_axis_name="c", subcore_axis_name="s")
    scratch = [pltpu.VMEM((per_w,), jnp.int32) for _ in range(9)]
    scratch += [pltpu.VMEM((per_w,), jnp.int32)]
    scratch += [pltpu.VMEM((CHUNK, H), jnp.float32) for _ in range(DEPTH)]
    scratch += [pltpu.SemaphoreType.DMA for _ in range(2 * DEPTH + 1)]
    fn = pl.kernel(
        functools.partial(_sc_body, n_tok),
        out_type=jax.ShapeDtypeStruct((n_tok, H), jnp.float32),
        mesh=mesh,
        scratch_types=scratch,
        interpret=interpret,
    )
    return fn(*idxs, table)


def kernel(mix, falsetto, breathy, bubble, strong, weak, pharyngeal,
           vibrato, glissando,
           W_mix, W_falsetto, W_breathy, W_bubble, W_strong, W_weak,
           W_pharyngeal, W_vibrato, W_glissando):
    b, t = mix.shape
    idxs = [a.reshape(-1) for a in
            (mix, falsetto, breathy, bubble, strong, weak, pharyngeal,
             vibrato, glissando)]
    ws = (W_mix, W_falsetto, W_breathy, W_bubble, W_strong, W_weak,
          W_pharyngeal, W_vibrato, W_glissando)
    table = _build_table(ws)
    out = _sc_lookup(idxs, table)
    return out.reshape(b, t, H)


# speedup vs baseline: 38.0383x; 1.2831x over previous
import functools
import jax
import jax.numpy as jnp
from jax import lax
from jax.experimental import pallas as pl
from jax.experimental.pallas import tpu as pltpu
from jax.experimental.pallas import tpu_sc as plsc

H = 256
NHI = 81
BAND = 256
NW = 32
CHUNK = 64
DEPTH = 4
AHEAD = 2


def _tbl_body(w0, w1, w2, w3, w4, w5, w6, w7, w8, out):
    i = pl.program_id(0)
    acc = w0[...]
    for wref in (w1, w2, w3, w4):
        w = wref[...]
        acc = jnp.concatenate(
            [acc + w[0:1, :], acc + w[1:2, :], acc + w[2:3, :]], axis=0)
    hi = jnp.zeros((1, H), jnp.float32)
    r = i
    for wref in (w6, w7, w8):
        w = wref[...]
        d = r % 3
        r = r // 3
        hi = hi + jnp.where(d == 0, w[0:1, :],
                            jnp.where(d == 1, w[1:2, :], w[2:3, :]))
    acc = acc + hi
    w5v = w5[...]
    pad = jnp.zeros((BAND - 243, H), jnp.float32)
    bands = []
    for k in range(3):
        bands += [acc + w5v[k:k + 1, :], pad]
    out[...] = jnp.concatenate(bands, axis=0) * 16.0


def _build_table(ws, interpret=False):
    return pl.pallas_call(
        _tbl_body,
        grid=(NHI // 3,),
        in_specs=[pl.BlockSpec((3, H), lambda i: (0, 0))] * 9,
        out_specs=pl.BlockSpec((3 * BAND, H), lambda i: (i, 0)),
        out_shape=jax.ShapeDtypeStruct((NHI * BAND, H), jnp.float32),
        interpret=interpret,
    )(*ws)


def _sc_body(n_tok, i0, i1, i2, i3, i4, i5, i6, i7, i8, table, out,
             b0, b1, b2, b3, b4, b5, b6, b7, b8, codes,
             rows0, rows1, rows2, rows3,
             isem, gsem0, gsem1, gsem2, gsem3, ssem0, ssem1, ssem2, ssem3):
    per_w = n_tok // NW
    nchunk = per_w // CHUNK
    wid = lax.axis_index("s") * 2 + lax.axis_index("c")
    base = wid * per_w
    t_len = i0.shape[1]
    rows_per_w = per_w // t_len
    row0 = wid * rows_per_w
    bufs = (b0, b1, b2, b3, b4, b5, b6, b7, b8)
    stages = []
    for src, dst in zip((i0, i1, i2, i3, i4, i5, i6, i7, i8), bufs):
        for r in range(rows_per_w):
            stages.append(pltpu.async_copy(
                src.at[row0 + r], dst.at[pl.ds(r * t_len, t_len)], isem))
    for h in stages:
        h.wait()

    rows = (rows0, rows1, rows2, rows3)
    gsem = (gsem0, gsem1, gsem2, gsem3)
    ssem = (ssem0, ssem1, ssem2, ssem3)

    def code_chunk(g):
        for j in range(CHUNK // 16):
            o = g * CHUNK + j * 16
            d = [b[pl.ds(o, 16)] for b in bufs]
            hi = ((d[8] * 3 + d[7]) * 3 + d[6]) * 3 + d[5]
            lo = (((d[4] * 3 + d[3]) * 3 + d[2]) * 3 + d[1]) * 3 + d[0]
            codes[pl.ds(o, 16)] = hi * BAND + lo

    def gather_start(g, s):
        pltpu.async_copy(table.at[codes.at[pl.ds(g * CHUNK, CHUNK)]],
                         rows[s], gsem[s])

    def gather_wait(g, s):
        pltpu.make_async_copy(table.at[codes.at[pl.ds(g * CHUNK, CHUNK)]],
                              rows[s], gsem[s]).wait()

    def scatter_start(g, s):
        pltpu.async_copy(rows[s], out.at[pl.ds(base + g * CHUNK, CHUNK)],
                         ssem[s])

    def scatter_wait(g, s):
        pltpu.make_async_copy(rows[s], out.at[pl.ds(base + g * CHUNK, CHUNK)],
                              ssem[s]).wait()

    for g in range(AHEAD):
        code_chunk(g)
        gather_start(g, g % DEPTH)

    @pl.loop(0, nchunk, step=DEPTH)
    def chunk_loop(k):
        for b in range(DEPTH):
            g = k + b
            s = b
            gather_wait(g, s)
            scatter_start(g, s)
            nxt = g + AHEAD
            s2 = (b + AHEAD) % DEPTH

            @pl.when(nxt < nchunk)
            def _():
                code_chunk(nxt)

                @pl.when(nxt - DEPTH >= 0)
                def _():
                    scatter_wait(nxt - DEPTH, s2)

                gather_start(nxt, s2)

    for s in range(DEPTH):
        scatter_wait(nchunk - DEPTH + s, s)


def _sc_lookup(idxs, table, interpret=False):
    n_tok = idxs[0].shape[0] * idxs[0].shape[1]
    per_w = n_tok // NW
    mesh = plsc.VectorSubcoreMesh(core_axis_name="c", subcore_axis_name="s")
    scratch = [pltpu.VMEM((per_w,), jnp.int32) for _ in range(9)]
    scratch += [pltpu.VMEM((per_w,), jnp.int32)]
    scratch += [pltpu.VMEM((CHUNK, H), jnp.float32) for _ in range(DEPTH)]
    scratch += [pltpu.SemaphoreType.DMA for _ in range(2 * DEPTH + 1)]
    fn = pl.kernel(
        functools.partial(_sc_body, n_tok),
        out_type=jax.ShapeDtypeStruct((n_tok, H), jnp.float32),
        mesh=mesh,
        scratch_types=scratch,
        interpret=interpret,
    )
    return fn(*idxs, table)


def kernel(mix, falsetto, breathy, bubble, strong, weak, pharyngeal,
           vibrato, glissando,
           W_mix, W_falsetto, W_breathy, W_bubble, W_strong, W_weak,
           W_pharyngeal, W_vibrato, W_glissando):
    b, t = mix.shape
    idxs = [mix, falsetto, breathy, bubble, strong, weak, pharyngeal,
            vibrato, glissando]
    ws = (W_mix, W_falsetto, W_breathy, W_bubble, W_strong, W_weak,
          W_pharyngeal, W_vibrato, W_glissando)
    table = _build_table(ws)
    out = _sc_lookup(idxs, table)
    return out.reshape(b, t, H)
